# initial kernel scaffold (unmeasured)
import jax
import jax.numpy as jnp
from jax import lax
from jax.experimental import pallas as pl
from jax.experimental.pallas import tpu as pltpu

N_Z = 4
B, S_LOC, H, D = 4, 256, 16, 64
BH = B * H
S_GLB = N_Z * S_LOC
SCALE = D ** -0.5


def kernel(Q, K, V):
    Qt = Q.transpose(0, 2, 1, 3).reshape(BH, S_LOC, D)
    Kt = K.transpose(0, 2, 1, 3).reshape(BH, S_LOC, D)
    Vt = V.transpose(0, 2, 1, 3).reshape(BH, S_LOC, D)

    def body(q_ref, k_ref, v_ref, out_ref, k_all, v_all, s_scr,
             send_sems, recv_sems):
        my_x = lax.axis_index("x")
        my_y = lax.axis_index("y")
        my_z = lax.axis_index("z")
        right = (my_x, my_y, (my_z + 1) % N_Z)
        left = (my_x, my_y, (my_z - 1) % N_Z)

        barrier = pltpu.get_barrier_semaphore()
        for nbr in (left, right):
            pl.semaphore_signal(barrier, inc=1, device_id=nbr,
                                device_id_type=pl.DeviceIdType.MESH)
        pl.semaphore_wait(barrier, 2)

        k_all[0] = k_ref[...]
        v_all[0] = v_ref[...]

        for h in range(N_Z - 1):
            rk = pltpu.make_async_remote_copy(
                src_ref=k_all.at[h], dst_ref=k_all.at[h + 1],
                send_sem=send_sems.at[0, h], recv_sem=recv_sems.at[0, h],
                device_id=right, device_id_type=pl.DeviceIdType.MESH)
            rv = pltpu.make_async_remote_copy(
                src_ref=v_all.at[h], dst_ref=v_all.at[h + 1],
                send_sem=send_sems.at[1, h], recv_sem=recv_sems.at[1, h],
                device_id=left, device_id_type=pl.DeviceIdType.MESH)
            rk.start()
            rv.start()
            rk.wait()
            rv.wait()

        def compute_bh(bh, carry):
            q = q_ref[bh]
            for s in range(N_Z):
                k = k_all[s, bh]
                s_scr[:, s * S_LOC:(s + 1) * S_LOC] = lax.dot_general(
                    q, k, (((1,), (1,)), ((), ())),
                    preferred_element_type=jnp.float32)
            s_full = s_scr[...] * SCALE
            m = jnp.max(s_full, axis=1, keepdims=True)
            p = jnp.exp(s_full - m)
            l = jnp.sum(p, axis=1, keepdims=True)
            o = jnp.zeros((S_LOC, D), jnp.float32)
            for s in range(N_Z):
                vs = (N_Z - s) % N_Z
                o = o + jnp.dot(p[:, s * S_LOC:(s + 1) * S_LOC],
                                v_all[vs, bh],
                                preferred_element_type=jnp.float32)
            out_ref[bh] = o / l
            return carry

        lax.fori_loop(0, BH, compute_bh, 0)

    out = pl.pallas_call(
        body,
        out_shape=jax.ShapeDtypeStruct((BH, S_LOC, D), jnp.float32),
        in_specs=[pl.BlockSpec(memory_space=pltpu.VMEM)] * 3,
        out_specs=pl.BlockSpec(memory_space=pltpu.VMEM),
        scratch_shapes=[
            pltpu.VMEM((N_Z, BH, S_LOC, D), jnp.float32),
            pltpu.VMEM((N_Z, BH, S_LOC, D), jnp.float32),
            pltpu.VMEM((S_LOC, S_GLB), jnp.float32),
            pltpu.SemaphoreType.DMA((2, N_Z - 1)),
            pltpu.SemaphoreType.DMA((2, N_Z - 1)),
        ],
        compiler_params=pltpu.CompilerParams(collective_id=0),
    )(Qt, Kt, Vt)

    return out.reshape(B, H, S_LOC, D).transpose(0, 2, 1, 3)


# baseline (device time: 342508 ns/iter reference)
import jax
import jax.numpy as jnp
from jax import lax
from jax.experimental import pallas as pl
from jax.experimental.pallas import tpu as pltpu

N_Z = 4
B, S_LOC, H, D = 4, 256, 16, 64
BH = B * H
S_GLB = N_Z * S_LOC
SCALE = D ** -0.5


def kernel(Q, K, V):
    Qt = Q.transpose(0, 2, 1, 3).reshape(BH, S_LOC, D)
    Kt = K.transpose(0, 2, 3, 1).reshape(BH, D, S_LOC)
    Vt = V.transpose(0, 2, 3, 1).reshape(BH, D, S_LOC)

    def body(q_ref, k_ref, v_ref, out_ref, k_all, v_all, s_scr,
             send_sems, recv_sems):
        my_x = lax.axis_index("x")
        my_y = lax.axis_index("y")
        my_z = lax.axis_index("z")
        right = (my_x, my_y, (my_z + 1) % N_Z)
        left = (my_x, my_y, (my_z - 1) % N_Z)

        barrier = pltpu.get_barrier_semaphore()
        for nbr in (left, right):
            pl.semaphore_signal(barrier, inc=1, device_id=nbr,
                                device_id_type=pl.DeviceIdType.MESH)
        pl.semaphore_wait(barrier, 2)

        for h in range(N_Z - 1):
            rk = pltpu.make_async_remote_copy(
                src_ref=k_ref if h == 0 else k_all.at[h - 1],
                dst_ref=k_all.at[h],
                send_sem=send_sems.at[0, h], recv_sem=recv_sems.at[0, h],
                device_id=right, device_id_type=pl.DeviceIdType.MESH)
            rv = pltpu.make_async_remote_copy(
                src_ref=v_ref if h == 0 else v_all.at[h - 1],
                dst_ref=v_all.at[h],
                send_sem=send_sems.at[1, h], recv_sem=recv_sems.at[1, h],
                device_id=left, device_id_type=pl.DeviceIdType.MESH)
            rk.start()
            rv.start()
            rk.wait()
            rv.wait()

        def compute_bh(bh, carry):
            q = q_ref[bh]
            for s in range(N_Z):
                kT = k_ref[bh] if s == 0 else k_all[s - 1, bh]
                s_scr[:, s * S_LOC:(s + 1) * S_LOC] = lax.dot_general(
                    q, kT, (((1,), (0,)), ((), ())),
                    preferred_element_type=jnp.float32)
            s_full = s_scr[...] * SCALE
            m = jnp.max(s_full, axis=1, keepdims=True)
            p = jnp.exp(s_full - m)
            l = jnp.sum(p, axis=1, keepdims=True)
            oT = jnp.zeros((D, S_LOC), jnp.float32)
            for s in range(N_Z):
                vs = (N_Z - s) % N_Z
                vT = v_ref[bh] if vs == 0 else v_all[vs - 1, bh]
                oT = oT + lax.dot_general(
                    vT, p[:, s * S_LOC:(s + 1) * S_LOC],
                    (((1,), (1,)), ((), ())),
                    preferred_element_type=jnp.float32)
            out_ref[bh] = oT / l.reshape(1, S_LOC)
            return carry

        lax.fori_loop(0, BH, compute_bh, 0)

    out = pl.pallas_call(
        body,
        out_shape=jax.ShapeDtypeStruct((BH, D, S_LOC), jnp.float32),
        in_specs=[pl.BlockSpec(memory_space=pltpu.VMEM)] * 3,
        out_specs=pl.BlockSpec(memory_space=pltpu.VMEM),
        scratch_shapes=[
            pltpu.VMEM((N_Z - 1, BH, D, S_LOC), jnp.float32),
            pltpu.VMEM((N_Z - 1, BH, D, S_LOC), jnp.float32),
            pltpu.VMEM((S_LOC, S_GLB), jnp.float32),
            pltpu.SemaphoreType.DMA((2, N_Z - 1)),
            pltpu.SemaphoreType.DMA((2, N_Z - 1)),
        ],
        compiler_params=pltpu.CompilerParams(
            collective_id=0, vmem_limit_bytes=38 * 1024 * 1024),
    )(Qt, Kt, Vt)

    return out.reshape(B, H, D, S_LOC).transpose(0, 3, 1, 2)


# device time: 212369 ns/iter; 1.6128x vs baseline; 1.6128x over previous
import jax
import jax.numpy as jnp
from jax import lax
from jax.experimental import pallas as pl
from jax.experimental.pallas import tpu as pltpu

N_Z = 4
B, S_LOC, H, D = 4, 256, 16, 64
BH = B * H
S_GLB = N_Z * S_LOC
SCALE = D ** -0.5

K_T, V_T = 0, 1
RIGHT, LEFT = 0, 1


def kernel(Q, K, V):
    Qt = Q.transpose(0, 2, 1, 3).reshape(BH, S_LOC, D).astype(jnp.bfloat16)
    Kt = K.transpose(0, 2, 3, 1).reshape(BH, D, S_LOC).astype(jnp.bfloat16)
    Vt = V.transpose(0, 2, 3, 1).reshape(BH, D, S_LOC).astype(jnp.bfloat16)

    def body(q_ref, k_ref, v_ref, out_ref, k_all, v_all, s_scr,
             send_sems, recv_sems):
        my_x = lax.axis_index("x")
        my_y = lax.axis_index("y")
        my_z = lax.axis_index("z")
        right = (my_x, my_y, my_z + 1)
        left = (my_x, my_y, my_z - 1)
        has_r = my_z < N_Z - 1
        has_l = my_z > 0

        k_all[my_z] = k_ref[...]
        v_all[my_z] = v_ref[...]

        barrier = pltpu.get_barrier_semaphore()

        @pl.when(has_l)
        def _():
            pl.semaphore_signal(barrier, inc=1, device_id=left,
                                device_id_type=pl.DeviceIdType.MESH)

        @pl.when(has_r)
        def _():
            pl.semaphore_signal(barrier, inc=1, device_id=right,
                                device_id_type=pl.DeviceIdType.MESH)

        pl.semaphore_wait(barrier, has_l.astype(jnp.int32)
                          + has_r.astype(jnp.int32))

        def rdma(t_buf, t_idx, origin, direction, hop, target):
            buf = k_all if t_buf == "k" else v_all
            return pltpu.make_async_remote_copy(
                src_ref=buf.at[origin], dst_ref=buf.at[origin],
                send_sem=send_sems.at[t_idx, direction, hop],
                recv_sem=recv_sems.at[t_idx, direction, hop],
                device_id=target, device_id_type=pl.DeviceIdType.MESH)

        for h in range(N_Z - 1):
            g_sr = has_r & (my_z - h >= 0)
            g_sl = has_l & (my_z + h <= N_Z - 1)
            g_rl = my_z - 1 - h >= 0
            g_rr = my_z + 1 + h <= N_Z - 1

            @pl.when(g_sr)
            def _(h=h):
                rdma("k", K_T, my_z - h, RIGHT, h, right).start()
                rdma("v", V_T, my_z - h, RIGHT, h, right).start()

            @pl.when(g_sl)
            def _(h=h):
                rdma("k", K_T, my_z + h, LEFT, h, left).start()
                rdma("v", V_T, my_z + h, LEFT, h, left).start()

            @pl.when(g_rl)
            def _(h=h):
                rdma("k", K_T, my_z - 1 - h, RIGHT, h, right).wait_recv()
                rdma("v", V_T, my_z - 1 - h, RIGHT, h, right).wait_recv()

            @pl.when(g_rr)
            def _(h=h):
                rdma("k", K_T, my_z + 1 + h, LEFT, h, left).wait_recv()
                rdma("v", V_T, my_z + 1 + h, LEFT, h, left).wait_recv()

        for h in range(N_Z - 1):
            g_sr = has_r & (my_z - h >= 0)
            g_sl = has_l & (my_z + h <= N_Z - 1)

            @pl.when(g_sr)
            def _(h=h):
                rdma("k", K_T, my_z - h, RIGHT, h, right).wait_send()
                rdma("v", V_T, my_z - h, RIGHT, h, right).wait_send()

            @pl.when(g_sl)
            def _(h=h):
                rdma("k", K_T, my_z + h, LEFT, h, left).wait_send()
                rdma("v", V_T, my_z + h, LEFT, h, left).wait_send()

        def compute_bh(bh, carry):
            q = q_ref[bh]
            for o in range(N_Z):
                s_scr[:, o * S_LOC:(o + 1) * S_LOC] = lax.dot_general(
                    q, k_all[o, bh], (((1,), (0,)), ((), ())),
                    preferred_element_type=jnp.float32)
            s_full = s_scr[...] * SCALE
            m = jnp.max(s_full, axis=1, keepdims=True)
            p = jnp.exp(s_full - m)
            l = jnp.sum(p, axis=1, keepdims=True)
            p_bf = p.astype(jnp.bfloat16)
            oT = jnp.zeros((D, S_LOC), jnp.float32)
            for o in range(N_Z):
                oT = oT + lax.dot_general(
                    v_all[o, bh], p_bf[:, o * S_LOC:(o + 1) * S_LOC],
                    (((1,), (1,)), ((), ())),
                    preferred_element_type=jnp.float32)
            out_ref[bh] = oT / l.reshape(1, S_LOC)
            return carry

        lax.fori_loop(0, BH, compute_bh, 0)

    out = pl.pallas_call(
        body,
        out_shape=jax.ShapeDtypeStruct((BH, D, S_LOC), jnp.float32),
        in_specs=[pl.BlockSpec(memory_space=pltpu.VMEM)] * 3,
        out_specs=pl.BlockSpec(memory_space=pltpu.VMEM),
        scratch_shapes=[
            pltpu.VMEM((N_Z, BH, D, S_LOC), jnp.bfloat16),
            pltpu.VMEM((N_Z, BH, D, S_LOC), jnp.bfloat16),
            pltpu.VMEM((S_LOC, S_GLB), jnp.float32),
            pltpu.SemaphoreType.DMA((2, 2, N_Z - 1)),
            pltpu.SemaphoreType.DMA((2, 2, N_Z - 1)),
        ],
        compiler_params=pltpu.CompilerParams(
            collective_id=0, vmem_limit_bytes=48 * 1024 * 1024),
    )(Qt, Kt, Vt)

    return out.reshape(B, H, D, S_LOC).transpose(0, 3, 1, 2)


# device time: 158682 ns/iter; 2.1585x vs baseline; 1.3383x over previous
import jax
import jax.numpy as jnp
from jax import lax
from jax.experimental import pallas as pl
from jax.experimental.pallas import tpu as pltpu

N_Z = 4
B, S_LOC, H, D = 4, 256, 16, 64
BH = B * H
QTR = BH // 4
S_GLB = N_Z * S_LOC
SCALE = D ** -0.5

ZR, ZL = 0, 1


def kernel(Q, K, V):
    Qt = Q.transpose(0, 2, 1, 3).reshape(BH, S_LOC, D).astype(jnp.bfloat16)
    Kt = K.transpose(0, 2, 3, 1).reshape(BH, D, S_LOC).astype(jnp.bfloat16)
    Vt = V.transpose(0, 2, 3, 1).reshape(BH, D, S_LOC).astype(jnp.bfloat16)
    KVt = jnp.stack([Kt, Vt], axis=1)

    def body(q_ref, kv_ref, out_ref, kv_all, s_scr,
             send_z, recv_z, send_sq, recv_sq):
        my_x = lax.axis_index("x")
        my_y = lax.axis_index("y")
        my_z = lax.axis_index("z")
        right = (my_x, my_y, my_z + 1)
        left = (my_x, my_y, my_z - 1)
        x_nbr = (1 - my_x, my_y, my_z)
        y_nbr = (my_x, 1 - my_y, my_z)
        has_r = my_z < N_Z - 1
        has_l = my_z > 0
        qid = my_x + 2 * my_y
        q_yn = my_x + 2 * (1 - my_y)
        row0 = 32 * my_y

        kv_all[my_z] = kv_ref[...]

        barrier = pltpu.get_barrier_semaphore()
        for nbr in (x_nbr, y_nbr):
            pl.semaphore_signal(barrier, inc=1, device_id=nbr,
                                device_id_type=pl.DeviceIdType.MESH)

        @pl.when(has_l)
        def _():
            pl.semaphore_signal(barrier, inc=1, device_id=left,
                                device_id_type=pl.DeviceIdType.MESH)

        @pl.when(has_r)
        def _():
            pl.semaphore_signal(barrier, inc=1, device_id=right,
                                device_id_type=pl.DeviceIdType.MESH)

        pl.semaphore_wait(barrier, 2 + has_l.astype(jnp.int32)
                          + has_r.astype(jnp.int32))

        def zcopy(origin, d, hop, target):
            return pltpu.make_async_remote_copy(
                src_ref=kv_all.at[origin, pl.ds(QTR * qid, QTR)],
                dst_ref=kv_all.at[origin, pl.ds(QTR * qid, QTR)],
                send_sem=send_z.at[d, hop], recv_sem=recv_z.at[d, hop],
                device_id=target, device_id_type=pl.DeviceIdType.MESH)

        def sqcopy(origin, start, size, hop, side, slot, target):
            return pltpu.make_async_remote_copy(
                src_ref=kv_all.at[origin, pl.ds(start, size)],
                dst_ref=kv_all.at[origin, pl.ds(start, size)],
                send_sem=send_sq.at[hop, side, slot],
                recv_sem=recv_sq.at[hop, side, slot],
                device_id=target, device_id_type=pl.DeviceIdType.MESH)

        def stage1(origin, hop, side):
            if (hop + side) % 2 == 0:
                sqcopy(origin, QTR * qid, QTR, hop, side, 0, x_nbr).start()
            else:
                sqcopy(origin, QTR * qid, QTR, hop, side, 0, y_nbr).start()

        def stage2(origin, hop, side):
            sqcopy(origin, QTR * qid, QTR, hop, side, 0,
                   x_nbr if (hop + side) % 2 == 0 else y_nbr).wait_recv()
            if (hop + side) % 2 == 0:
                sqcopy(origin, row0, 2 * QTR, hop, side, 1, y_nbr).start()
            else:
                sqcopy(origin, QTR * qid, QTR, hop, side, 1, x_nbr).start()
                sqcopy(origin, QTR * q_yn, QTR, hop, side, 2, x_nbr).start()

        def stage2_wait(origin, hop, side):
            if (hop + side) % 2 == 0:
                sqcopy(origin, row0, 2 * QTR, hop, side, 1, y_nbr).wait_recv()
            else:
                sqcopy(origin, QTR * qid, QTR, hop, side, 1, x_nbr).wait_recv()
                sqcopy(origin, QTR * q_yn, QTR, hop, side, 2, x_nbr).wait_recv()

        for h in range(N_Z - 1):
            g_sr = has_r & (my_z - h >= 0)
            g_sl = has_l & (my_z + h <= N_Z - 1)
            g_rl = my_z - 1 - h >= 0
            g_rr = my_z + 1 + h <= N_Z - 1

            @pl.when(g_sr)
            def _(h=h):
                zcopy(my_z - h, ZR, h, right).start()

            @pl.when(g_sl)
            def _(h=h):
                zcopy(my_z + h, ZL, h, left).start()

            @pl.when(g_rl)
            def _(h=h):
                zcopy(my_z - 1 - h, ZR, h, right).wait_recv()
                stage1(my_z - 1 - h, h, 0)

            @pl.when(g_rr)
            def _(h=h):
                zcopy(my_z + 1 + h, ZL, h, left).wait_recv()
                stage1(my_z + 1 + h, h, 1)

        for h in range(N_Z - 1):
            @pl.when(my_z - 1 - h >= 0)
            def _(h=h):
                stage2(my_z - 1 - h, h, 0)

            @pl.when(my_z + 1 + h <= N_Z - 1)
            def _(h=h):
                stage2(my_z + 1 + h, h, 1)

        for h in range(N_Z - 1):
            @pl.when(my_z - 1 - h >= 0)
            def _(h=h):
                stage2_wait(my_z - 1 - h, h, 0)

            @pl.when(my_z + 1 + h <= N_Z - 1)
            def _(h=h):
                stage2_wait(my_z + 1 + h, h, 1)

            @pl.when(has_r & (my_z - h >= 0))
            def _(h=h):
                zcopy(my_z - h, ZR, h, right).wait_send()

            @pl.when(has_l & (my_z + h <= N_Z - 1))
            def _(h=h):
                zcopy(my_z + h, ZL, h, left).wait_send()

            @pl.when(my_z - 1 - h >= 0)
            def _(h=h):
                o = my_z - 1 - h
                t1 = x_nbr if h % 2 == 0 else y_nbr
                sqcopy(o, QTR * qid, QTR, h, 0, 0, t1).wait_send()
                if h % 2 == 0:
                    sqcopy(o, row0, 2 * QTR, h, 0, 1, y_nbr).wait_send()
                else:
                    sqcopy(o, QTR * qid, QTR, h, 0, 1, x_nbr).wait_send()
                    sqcopy(o, QTR * q_yn, QTR, h, 0, 2, x_nbr).wait_send()

            @pl.when(my_z + 1 + h <= N_Z - 1)
            def _(h=h):
                o = my_z + 1 + h
                t1 = x_nbr if (h + 1) % 2 == 0 else y_nbr
                sqcopy(o, QTR * qid, QTR, h, 1, 0, t1).wait_send()
                if (h + 1) % 2 == 0:
                    sqcopy(o, row0, 2 * QTR, h, 1, 1, y_nbr).wait_send()
                else:
                    sqcopy(o, QTR * qid, QTR, h, 1, 1, x_nbr).wait_send()
                    sqcopy(o, QTR * q_yn, QTR, h, 1, 2, x_nbr).wait_send()

        def compute_bh(bh, carry):
            q = q_ref[bh]
            for o in range(N_Z):
                s_scr[:, o * S_LOC:(o + 1) * S_LOC] = lax.dot_general(
                    q, kv_all[o, bh, 0], (((1,), (0,)), ((), ())),
                    preferred_element_type=jnp.float32)
            s_full = s_scr[...] * SCALE
            m = jnp.max(s_full, axis=1, keepdims=True)
            p = jnp.exp(s_full - m)
            l = jnp.sum(p, axis=1, keepdims=True)
            p_bf = p.astype(jnp.bfloat16)
            oT = jnp.zeros((D, S_LOC), jnp.float32)
            for o in range(N_Z):
                oT = oT + lax.dot_general(
                    kv_all[o, bh, 1], p_bf[:, o * S_LOC:(o + 1) * S_LOC],
                    (((1,), (1,)), ((), ())),
                    preferred_element_type=jnp.float32)
            out_ref[bh] = oT / l.reshape(1, S_LOC)
            return carry

        lax.fori_loop(0, BH, compute_bh, 0)

    out = pl.pallas_call(
        body,
        out_shape=jax.ShapeDtypeStruct((BH, D, S_LOC), jnp.float32),
        in_specs=[pl.BlockSpec(memory_space=pltpu.VMEM)] * 2,
        out_specs=pl.BlockSpec(memory_space=pltpu.VMEM),
        scratch_shapes=[
            pltpu.VMEM((N_Z, BH, 2, D, S_LOC), jnp.bfloat16),
            pltpu.VMEM((S_LOC, S_GLB), jnp.float32),
            pltpu.SemaphoreType.DMA((2, N_Z - 1)),
            pltpu.SemaphoreType.DMA((2, N_Z - 1)),
            pltpu.SemaphoreType.DMA((N_Z - 1, 2, 3)),
            pltpu.SemaphoreType.DMA((N_Z - 1, 2, 3)),
        ],
        compiler_params=pltpu.CompilerParams(
            collective_id=0, vmem_limit_bytes=48 * 1024 * 1024),
    )(Qt, KVt)

    return out.reshape(B, H, D, S_LOC).transpose(0, 3, 1, 2)
